# baseline (speedup 1.0000x reference)
import functools

import jax
import jax.numpy as jnp
from jax import lax
from jax.experimental import pallas as pl
from jax.experimental.pallas import tpu as pltpu
from jax.experimental.pallas import tpu_sc as plsc

ROWS = 8
NCHUNK = 192 // ROWS
LANES = 16
UNROLL = 8


def _sc_body(pt_hbm, pos_hbm, out_hbm, pos_v, ibuf, obuf, sem_in, sem_out):
    c = lax.axis_index("c")
    s = lax.axis_index("s")
    wid = s * 2 + c
    r0 = wid * ROWS

    B = pt_hbm.shape[0]
    P = pt_hbm.shape[2]

    def in_copy(b, slot):
        return pltpu.make_async_copy(
            pt_hbm.at[b, pl.ds(r0, ROWS), :], ibuf.at[slot], sem_in.at[slot]
        )

    def out_copy(b, slot):
        return pltpu.make_async_copy(
            obuf.at[slot], out_hbm.at[b, pl.ds(r0, ROWS), :], sem_out.at[slot]
        )

    @pl.when(wid < NCHUNK)
    def _():
        pltpu.sync_copy(pos_hbm.at[pl.ds(r0, ROWS), :], pos_v)
        in_copy(0, 0).start()
        in_copy(1, 1).start()

        def pair_body(i, carry):
            b0 = 2 * i
            in_copy(b0, 0).wait()
            in_copy(b0 + 1, 1).wait()

            @pl.when(i > 0)
            def _():
                out_copy(b0 - 2, 0).wait()
                out_copy(b0 - 1, 1).wait()

            for row in range(ROWS):

                def col_body(j, acc, row=row):
                    for u in range(UNROLL):
                        sl = pl.ds((j * UNROLL + u) * LANES, LANES)
                        pv = pos_v[row, sl]
                        obuf[0, row, sl] = ibuf[0, row, sl] + pv
                        obuf[1, row, sl] = ibuf[1, row, sl] + pv
                    return acc

                lax.fori_loop(0, P // (LANES * UNROLL), col_body, 0)

            out_copy(b0, 0).start()
            out_copy(b0 + 1, 1).start()

            @pl.when(b0 + 2 < B)
            def _():
                in_copy(b0 + 2, 0).start()
                in_copy(b0 + 3, 1).start()

            return carry

        lax.fori_loop(0, B // 2, pair_body, 0)
        out_copy(B - 2, 0).wait()
        out_copy(B - 1, 1).wait()


def kernel(patch, position_embedding):
    B, P, D = patch.shape
    pt = jnp.transpose(patch, (0, 2, 1))
    post = jnp.transpose(position_embedding, (1, 0))
    mesh = plsc.VectorSubcoreMesh(core_axis_name="c", subcore_axis_name="s")
    run = functools.partial(
        pl.kernel,
        mesh=mesh,
        out_type=jax.ShapeDtypeStruct((B, D, P), patch.dtype),
        scratch_types=[
            pltpu.VMEM((ROWS, P), patch.dtype),
            pltpu.VMEM((2, ROWS, P), patch.dtype),
            pltpu.VMEM((2, ROWS, P), patch.dtype),
            pltpu.SemaphoreType.DMA((2,)),
            pltpu.SemaphoreType.DMA((2,)),
        ],
    )(_sc_body)
    out = run(pt, post)
    return jnp.transpose(out, (0, 2, 1))


# speedup vs baseline: 3.0225x; 3.0225x over previous
import jax
import jax.numpy as jnp
from jax.experimental import pallas as pl
from jax.experimental.pallas import tpu as pltpu

BB = 4
NBUF = 8


def _body(patch_hbm, pos_hbm, out_hbm, pos_v, ibufs, obufs, sem_pos,
          sem_in, sem_out):
    nchunk = patch_hbm.shape[0] // BB

    def in_copy(i):
        slot = i % NBUF
        return pltpu.make_async_copy(
            patch_hbm.at[pl.ds(i * BB, BB)], ibufs.at[slot], sem_in.at[slot]
        )

    def out_copy(i):
        slot = i % NBUF
        return pltpu.make_async_copy(
            obufs.at[slot], out_hbm.at[pl.ds(i * BB, BB)], sem_out.at[slot]
        )

    pos_copy = pltpu.make_async_copy(pos_hbm, pos_v, sem_pos)
    pos_copy.start()
    for i in range(NBUF):
        in_copy(i).start()
    pos_copy.wait()

    for i in range(nchunk):
        slot = i % NBUF
        in_copy(i).wait()
        if i >= NBUF:
            out_copy(i - NBUF).wait()
        obufs[slot] = ibufs[slot] + pos_v[...]
        out_copy(i).start()
        if i + NBUF < nchunk:
            in_copy(i + NBUF).start()

    for i in range(max(0, nchunk - NBUF), nchunk):
        out_copy(i).wait()


def kernel(patch, position_embedding):
    B, P, D = patch.shape
    pt = jnp.transpose(patch, (0, 2, 1))
    post = jnp.transpose(position_embedding, (1, 0))
    out = pl.pallas_call(
        _body,
        in_specs=[
            pl.BlockSpec(memory_space=pl.ANY),
            pl.BlockSpec(memory_space=pl.ANY),
        ],
        out_specs=pl.BlockSpec(memory_space=pl.ANY),
        out_shape=jax.ShapeDtypeStruct((B, D, P), patch.dtype),
        scratch_shapes=[
            pltpu.VMEM((D, P), patch.dtype),
            pltpu.VMEM((NBUF, BB, D, P), patch.dtype),
            pltpu.VMEM((NBUF, BB, D, P), patch.dtype),
            pltpu.SemaphoreType.DMA,
            pltpu.SemaphoreType.DMA((NBUF,)),
            pltpu.SemaphoreType.DMA((NBUF,)),
        ],
    )(pt, post)
    return jnp.transpose(out, (0, 2, 1))


# speedup vs baseline: 3.1492x; 1.0419x over previous
import jax
import jax.numpy as jnp
from jax.experimental import pallas as pl
from jax.experimental.pallas import tpu as pltpu

BB = 8
NBUF = 4


def _body(patch_hbm, pos_hbm, out_hbm, pos_v, ibufs, obufs, sem_pos,
          sem_in, sem_out):
    nchunk = patch_hbm.shape[0] // BB

    def in_copy(i):
        slot = i % NBUF
        return pltpu.make_async_copy(
            patch_hbm.at[pl.ds(i * BB, BB)], ibufs.at[slot], sem_in.at[slot]
        )

    def out_copy(i):
        slot = i % NBUF
        return pltpu.make_async_copy(
            obufs.at[slot], out_hbm.at[pl.ds(i * BB, BB)], sem_out.at[slot]
        )

    pos_copy = pltpu.make_async_copy(pos_hbm, pos_v, sem_pos)
    pos_copy.start()
    for i in range(NBUF):
        in_copy(i).start()
    pos_copy.wait()

    for i in range(nchunk):
        slot = i % NBUF
        in_copy(i).wait()
        if i >= NBUF:
            out_copy(i - NBUF).wait()
        obufs[slot] = ibufs[slot] + pos_v[...]
        out_copy(i).start()
        if i + NBUF < nchunk:
            in_copy(i + NBUF).start()

    for i in range(max(0, nchunk - NBUF), nchunk):
        out_copy(i).wait()


def kernel(patch, position_embedding):
    B, P, D = patch.shape
    pt = jnp.transpose(patch, (0, 2, 1))
    post = jnp.transpose(position_embedding, (1, 0))
    out = pl.pallas_call(
        _body,
        in_specs=[
            pl.BlockSpec(memory_space=pl.ANY),
            pl.BlockSpec(memory_space=pl.ANY),
        ],
        out_specs=pl.BlockSpec(memory_space=pl.ANY),
        out_shape=jax.ShapeDtypeStruct((B, D, P), patch.dtype),
        scratch_shapes=[
            pltpu.VMEM((D, P), patch.dtype),
            pltpu.VMEM((NBUF, BB, D, P), patch.dtype),
            pltpu.VMEM((NBUF, BB, D, P), patch.dtype),
            pltpu.SemaphoreType.DMA,
            pltpu.SemaphoreType.DMA((NBUF,)),
            pltpu.SemaphoreType.DMA((NBUF,)),
        ],
    )(pt, post)
    return jnp.transpose(out, (0, 2, 1))
